# baseline (speedup 1.0000x reference)
import functools

import jax
import jax.numpy as jnp
from jax import lax
from jax.experimental import pallas as pl
from jax.experimental.pallas import tpu as pltpu
from jax.experimental.pallas import tpu_sc as plsc

LATENT = 1024
WORD = 64
NWORDS = 1024
B = 4608
NTOK = B * (LATENT // WORD)

ROWS_PER_BLOCK = 4096
NBLOCKS = NTOK // ROWS_PER_BLOCK

_NC, _NS = 2, 16
_NW = _NC * _NS
_B_PER_W = NTOK // _NW
_NCHUNK = 3
_CHUNK = _B_PER_W // _NCHUNK


def _distance_argmin_body(z_ref, emb_ref, fiota_ref, idx_ref, loss_ref):
    z = z_ref[...]
    e = emb_ref[...]
    zsq = jnp.sum(z * z, axis=1, keepdims=True)
    esq = jnp.sum(e * e, axis=1)
    dot2 = lax.dot_general(
        z + z, e, dimension_numbers=(((1,), (1,)), ((), ())),
        preferred_element_type=jnp.float32)
    d = (zsq + esq[None, :]) - dot2
    mn = jnp.min(d, axis=1)
    idxf = jnp.min(jnp.where(d == mn[:, None], fiota_ref[...], 65536.0), axis=1)
    idx = idxf.astype(jnp.int32)
    idx_ref[...] = idx.reshape(idx_ref.shape)

    @pl.when(pl.program_id(0) == 0)
    def _():
        loss_ref[0, 0] = 0.0

    loss_ref[0, 0] += jnp.sum(mn)


def _tc_distance_argmin(z_flat, emb, base_block, nblocks):
    call = pl.pallas_call(
        _distance_argmin_body,
        grid=(nblocks,),
        in_specs=[
            pl.BlockSpec((ROWS_PER_BLOCK, WORD), lambda i: (i + base_block, 0)),
            pl.BlockSpec((NWORDS, WORD), lambda i: (0, 0)),
            pl.BlockSpec((1, NWORDS), lambda i: (0, 0)),
        ],
        out_specs=[
            pl.BlockSpec((ROWS_PER_BLOCK // 128, 128), lambda i: (i, 0)),
            pl.BlockSpec(memory_space=pltpu.SMEM),
        ],
        out_shape=[
            jax.ShapeDtypeStruct((nblocks * ROWS_PER_BLOCK // 128, 128),
                                 jnp.int32),
            jax.ShapeDtypeStruct((1, 1), jnp.float32),
        ],
    )
    fiota = lax.broadcasted_iota(jnp.float32, (1, NWORDS), 1)
    return call(z_flat, emb, fiota)


@functools.lru_cache(maxsize=4)
def _make_sc_gather(nrows):
    bpw = nrows // _NW
    nchunk = 2 if bpw <= 2048 else 3
    chunk = bpw // nchunk

    @functools.partial(
        pl.kernel,
        mesh=plsc.VectorSubcoreMesh(core_axis_name="c", subcore_axis_name="s"),
        out_type=jax.ShapeDtypeStruct((nrows, WORD), jnp.float32),
        scratch_types=[
            pltpu.VMEM((bpw,), jnp.int32),
            pltpu.VMEM((chunk, WORD), jnp.float32),
            pltpu.VMEM((chunk, WORD), jnp.float32),
            pltpu.SemaphoreType.DMA,
            pltpu.SemaphoreType.DMA,
            pltpu.SemaphoreType.DMA,
            pltpu.SemaphoreType.DMA,
        ],
        compiler_params=pltpu.CompilerParams(use_tc_tiling_on_sc=False),
    )
    def _sc_gather(emb_hbm, idx_hbm, out_hbm, idx_v, rows0, rows1,
                   sg0, sg1, ss0, ss1):
        wid = lax.axis_index("s") * _NC + lax.axis_index("c")
        base = wid * bpw
        pltpu.sync_copy(idx_hbm.at[pl.ds(base, bpw)], idx_v)
        rows = (rows0, rows1)
        gsem = (sg0, sg1)
        ssem = (ss0, ss1)
        gathers = [None] * nchunk
        scatters = [None] * nchunk
        for j in range(nchunk):
            p = j % 2
            if j >= 2:
                scatters[j - 2].wait()
            gathers[j] = pltpu.async_copy(
                emb_hbm.at[idx_v.at[pl.ds(j * chunk, chunk)]],
                rows[p], gsem[p])
            if j >= 1:
                gathers[j - 1].wait()
                scatters[j - 1] = pltpu.async_copy(
                    rows[1 - p], out_hbm.at[pl.ds(base + (j - 1) * chunk, chunk)],
                    ssem[1 - p])
        gathers[nchunk - 1].wait()
        scatters[nchunk - 1] = pltpu.async_copy(
            rows[(nchunk - 1) % 2],
            out_hbm.at[pl.ds(base + (nchunk - 1) * chunk, chunk)],
            ssem[(nchunk - 1) % 2])
        for j in range(max(0, nchunk - 2), nchunk):
            scatters[j].wait()

    return _sc_gather


_NSPLIT = 3


def kernel(image, emb):
    z_flat = image[0].reshape(NTOK, WORD)
    blocks_per = NBLOCKS // _NSPLIT
    rows_per = blocks_per * ROWS_PER_BLOCK
    gather = _make_sc_gather(rows_per)
    lat_parts, loss_parts = [], []
    for k in range(_NSPLIT):
        idx2d, loss_sum = _tc_distance_argmin(
            z_flat, emb, k * blocks_per, blocks_per)
        z_q = gather(emb, idx2d.reshape(rows_per))
        lat_parts.append(z_q.reshape(rows_per // (LATENT // WORD), LATENT))
        loss_parts.append(loss_sum[0, 0])
    latent = jnp.concatenate(lat_parts, axis=0)
    loss = (3.5 / float(NTOK * WORD)) * sum(loss_parts)
    return latent, latent, loss


# speedup vs baseline: 1.0119x; 1.0119x over previous
import functools

import jax
import jax.numpy as jnp
from jax import lax
from jax.experimental import pallas as pl
from jax.experimental.pallas import tpu as pltpu
from jax.experimental.pallas import tpu_sc as plsc

LATENT = 1024
WORD = 64
NWORDS = 1024
B = 4608
NTOK = B * (LATENT // WORD)

ROWS_PER_BLOCK = 6144
NBLOCKS = NTOK // ROWS_PER_BLOCK

_NC, _NS = 2, 16
_NW = _NC * _NS
_B_PER_W = NTOK // _NW
_NCHUNK = 3
_CHUNK = _B_PER_W // _NCHUNK


def _distance_argmin_body(z_ref, emb_ref, fiota_ref, idx_ref, loss_ref):
    z = z_ref[...]
    e = emb_ref[...]
    zsq = jnp.sum(z * z, axis=1, keepdims=True)
    esq = jnp.sum(e * e, axis=1)
    dot2 = lax.dot_general(
        z + z, e, dimension_numbers=(((1,), (1,)), ((), ())),
        preferred_element_type=jnp.float32)
    d = (zsq + esq[None, :]) - dot2
    mn = jnp.min(d, axis=1)
    idxf = jnp.min(jnp.where(d == mn[:, None], fiota_ref[...], 65536.0), axis=1)
    idx = idxf.astype(jnp.int32)
    idx_ref[...] = idx.reshape(idx_ref.shape)

    @pl.when(pl.program_id(0) == 0)
    def _():
        loss_ref[0, 0] = 0.0

    loss_ref[0, 0] += jnp.sum(mn)


def _tc_distance_argmin(z_flat, emb, base_block, nblocks):
    call = pl.pallas_call(
        _distance_argmin_body,
        grid=(nblocks,),
        in_specs=[
            pl.BlockSpec((ROWS_PER_BLOCK, WORD), lambda i: (i + base_block, 0)),
            pl.BlockSpec((NWORDS, WORD), lambda i: (0, 0)),
            pl.BlockSpec((1, NWORDS), lambda i: (0, 0)),
        ],
        out_specs=[
            pl.BlockSpec((ROWS_PER_BLOCK // 128, 128), lambda i: (i, 0)),
            pl.BlockSpec(memory_space=pltpu.SMEM),
        ],
        out_shape=[
            jax.ShapeDtypeStruct((nblocks * ROWS_PER_BLOCK // 128, 128),
                                 jnp.int32),
            jax.ShapeDtypeStruct((1, 1), jnp.float32),
        ],
    )
    fiota = lax.broadcasted_iota(jnp.float32, (1, NWORDS), 1)
    return call(z_flat, emb, fiota)


@functools.lru_cache(maxsize=4)
def _make_sc_gather(nrows):
    bpw = nrows // _NW
    nchunk = 2 if bpw <= 2048 else 3
    chunk = bpw // nchunk

    @functools.partial(
        pl.kernel,
        mesh=plsc.VectorSubcoreMesh(core_axis_name="c", subcore_axis_name="s"),
        out_type=jax.ShapeDtypeStruct((nrows, WORD), jnp.float32),
        scratch_types=[
            pltpu.VMEM((bpw,), jnp.int32),
            pltpu.VMEM((chunk, WORD), jnp.float32),
            pltpu.VMEM((chunk, WORD), jnp.float32),
            pltpu.SemaphoreType.DMA,
            pltpu.SemaphoreType.DMA,
            pltpu.SemaphoreType.DMA,
            pltpu.SemaphoreType.DMA,
        ],
        compiler_params=pltpu.CompilerParams(use_tc_tiling_on_sc=False),
    )
    def _sc_gather(emb_hbm, idx_hbm, out_hbm, idx_v, rows0, rows1,
                   sg0, sg1, ss0, ss1):
        wid = lax.axis_index("s") * _NC + lax.axis_index("c")
        base = wid * bpw
        pltpu.sync_copy(idx_hbm.at[pl.ds(base, bpw)], idx_v)
        rows = (rows0, rows1)
        gsem = (sg0, sg1)
        ssem = (ss0, ss1)
        gathers = [None] * nchunk
        scatters = [None] * nchunk
        for j in range(nchunk):
            p = j % 2
            if j >= 2:
                scatters[j - 2].wait()
            gathers[j] = pltpu.async_copy(
                emb_hbm.at[idx_v.at[pl.ds(j * chunk, chunk)]],
                rows[p], gsem[p])
            if j >= 1:
                gathers[j - 1].wait()
                scatters[j - 1] = pltpu.async_copy(
                    rows[1 - p], out_hbm.at[pl.ds(base + (j - 1) * chunk, chunk)],
                    ssem[1 - p])
        gathers[nchunk - 1].wait()
        scatters[nchunk - 1] = pltpu.async_copy(
            rows[(nchunk - 1) % 2],
            out_hbm.at[pl.ds(base + (nchunk - 1) * chunk, chunk)],
            ssem[(nchunk - 1) % 2])
        for j in range(max(0, nchunk - 2), nchunk):
            scatters[j].wait()

    return _sc_gather


_NSPLIT = 2


def kernel(image, emb):
    z_flat = image[0].reshape(NTOK, WORD)
    blocks_per = NBLOCKS // _NSPLIT
    rows_per = blocks_per * ROWS_PER_BLOCK
    gather = _make_sc_gather(rows_per)
    lat_parts, loss_parts = [], []
    for k in range(_NSPLIT):
        idx2d, loss_sum = _tc_distance_argmin(
            z_flat, emb, k * blocks_per, blocks_per)
        z_q = gather(emb, idx2d.reshape(rows_per))
        lat_parts.append(z_q.reshape(rows_per // (LATENT // WORD), LATENT))
        loss_parts.append(loss_sum[0, 0])
    latent = jnp.concatenate(lat_parts, axis=0)
    loss = (3.5 / float(NTOK * WORD)) * sum(loss_parts)
    return latent, latent, loss
